# Initial kernel scaffold; baseline (speedup 1.0000x reference)
import jax
import jax.numpy as jnp
from jax.experimental import pallas as pl


def kernel(ids, num_samples, adj_info, features, W, b):
    raise NotImplementedError("write your pallas kernel here")


# speedup vs baseline: 5.5092x; 5.5092x over previous
import functools

import jax
import jax.numpy as jnp
from jax import lax
from jax.experimental import pallas as pl
from jax.experimental.pallas import tpu as pltpu
from jax.experimental.pallas import tpu_sc as plsc

V = 2048
N = 64
D = 512
S = 16
L = 16
NC = 2
NS = 16
NW = NC * NS
VW = V // NW
CH = D // L
HALF = 50000


def _tc_pair_call(adj_info):
    R2 = adj_info.shape[0]
    R = R2 // 2
    bo = 2000
    nblk = R // bo

    def body(a0_ref, a1_ref, o_ref):
        o_ref[:, 0:N] = a0_ref[:]
        o_ref[:, N:2 * N] = a1_ref[:]

    return pl.pallas_call(
        body,
        grid=(nblk,),
        in_specs=[
            pl.BlockSpec((bo, N), lambda i: (i, 0)),
            pl.BlockSpec((bo, N), lambda i, _n=nblk: (i + _n, 0)),
        ],
        out_specs=pl.BlockSpec((bo, 2 * N), lambda i: (i, 0)),
        out_shape=jax.ShapeDtypeStruct((R, 2 * N), jnp.int32),
    )(adj_info, adj_info)


def _sc_gather_call(ids_q, adj2, features):
    mesh = plsc.VectorSubcoreMesh(core_axis_name="c", subcore_axis_name="s")
    V4 = V // 4
    VW4 = V4 // NW

    @functools.partial(
        pl.kernel,
        mesh=mesh,
        out_type=(
            jax.ShapeDtypeStruct((V4, D), jnp.float32),
            jax.ShapeDtypeStruct((V4, 2 * N), jnp.int32),
        ),
        scratch_types=[
            pltpu.VMEM((VW4,), jnp.int32),
            pltpu.VMEM((VW4, D), jnp.float32),
            pltpu.VMEM((VW4, 2 * N), jnp.int32),
            pltpu.SemaphoreType.DMA,
            pltpu.SemaphoreType.DMA,
        ],
    )
    def k(ids_hbm, adj2_hbm, feat_hbm, vf_out, stage_out,
          ids_v, vf_v, stage_v, sem0, sem1):
        wid = lax.axis_index("s") * NC + lax.axis_index("c")
        base = wid * VW4
        pltpu.sync_copy(ids_hbm.at[pl.ds(base, VW4)], ids_v)
        c0 = pltpu.async_copy(feat_hbm.at[ids_v], vf_v, sem0)
        c0.wait()
        cc = ids_v[pl.ds(0, L)]
        ids_v[pl.ds(0, L)] = jnp.where(cc >= HALF, cc - HALF, cc)
        c1 = pltpu.async_copy(adj2_hbm.at[ids_v], stage_v, sem1)
        c1.wait()
        pltpu.sync_copy(vf_v, vf_out.at[pl.ds(base, VW4)])
        pltpu.sync_copy(stage_v, stage_out.at[pl.ds(base, VW4)])

    return k(ids_q, adj2, features)


def _tc_halve_call(stage, ids2):
    blk = 256

    def body(st_ref, id_ref, o_ref):
        m = id_ref[:] >= HALF
        x = st_ref[:]
        o_ref[:] = jnp.where(m, x[:, N:2 * N], x[:, 0:N])

    V4 = V // 4
    return pl.pallas_call(
        body,
        grid=(V4 // blk,),
        in_specs=[
            pl.BlockSpec((blk, 2 * N), lambda i: (i, 0)),
            pl.BlockSpec((blk, 1), lambda i: (i, 0)),
        ],
        out_specs=pl.BlockSpec((blk, N), lambda i: (i, 0)),
        out_shape=jax.ShapeDtypeStruct((V4, N), jnp.int32),
    )(stage, ids2)


def _tc_matmul_call(vf, W, b):
    blk = 512

    def body(vf_ref, w_ref, b_ref, o_ref):
        o_ref[:] = (
            jnp.dot(vf_ref[:], w_ref[:], preferred_element_type=jnp.float32)
            + b_ref[:]
        )

    V4 = V // 4
    return pl.pallas_call(
        body,
        grid=(V4 // blk,),
        in_specs=[
            pl.BlockSpec((blk, D), lambda i: (i, 0)),
            pl.BlockSpec((D, D), lambda i: (0, 0)),
            pl.BlockSpec((1, D), lambda i: (0, 0)),
        ],
        out_specs=pl.BlockSpec((blk, D), lambda i: (i, 0)),
        out_shape=jax.ShapeDtypeStruct((V4, D), jnp.float32),
    )(vf, W, b.reshape(1, D))


def _sc_scores_call(l, adjl, features):
    mesh = plsc.VectorSubcoreMesh(core_axis_name="c", subcore_axis_name="s")
    V4 = V // 4
    VW4 = V4 // NW

    @functools.partial(
        pl.kernel,
        mesh=mesh,
        out_type=jax.ShapeDtypeStruct((V4, N * L), jnp.float32),
        scratch_types=[
            pltpu.VMEM((VW4, N), jnp.int32),
            pltpu.VMEM((2 * VW4, N // 2), jnp.int32),
            pltpu.VMEM((VW4, D), jnp.float32),
            pltpu.VMEM((N // 2, D), jnp.float32),
            pltpu.VMEM((N // 2, D), jnp.float32),
            pltpu.VMEM((VW4, N * L), jnp.float32),
            pltpu.SemaphoreType.DMA,
            pltpu.SemaphoreType.DMA,
        ],
    )
    def k(l_hbm, adjl_hbm, feat_hbm, p_out,
          adj_v, adj_i, l_v, rows0, rows1, sc_v, sem0, sem1):
        wid = lax.axis_index("s") * NC + lax.axis_index("c")
        base = wid * VW4
        NH = N // 2
        HW = 2 * VW4

        pltpu.sync_copy(adjl_hbm.at[pl.ds(base, VW4)], adj_v)
        pltpu.sync_copy(l_hbm.at[pl.ds(base, VW4)], l_v)
        for i in range(VW4):
            for h2 in range(2):
                for c2 in range(2):
                    adj_i[2 * i + h2, pl.ds(c2 * L, L)] = (
                        adj_v[i, pl.ds(h2 * NH + c2 * L, L)])

        def compute(rows, h):
            i = h // 2
            hb = h % 2
            lrow = [l_v[i, pl.ds(c * L, L)] for c in range(CH)]

            def gbody(g, carry):
                for kk in range(L):
                    n = g * L + kk
                    acc = rows[n, pl.ds(0, L)] * lrow[0]
                    for c in range(1, CH):
                        acc = acc + rows[n, pl.ds(c * L, L)] * lrow[c]
                    sc_v[i, pl.ds((hb * NH + n) * L, L)] = acc
                return carry

            lax.fori_loop(0, NH // L, gbody, 0)

        pltpu.async_copy(feat_hbm.at[adj_i.at[0]], rows0, sem0)
        pltpu.async_copy(feat_hbm.at[adj_i.at[1]], rows1, sem1)

        def body(j, carry):
            h0 = 2 * j
            pltpu.make_async_copy(feat_hbm.at[adj_i.at[h0]], rows0, sem0).wait()
            compute(rows0, h0)

            @pl.when(h0 + 2 < HW)
            def _():
                pltpu.async_copy(feat_hbm.at[adj_i.at[h0 + 2]], rows0, sem0)

            pltpu.make_async_copy(
                feat_hbm.at[adj_i.at[h0 + 1]], rows1, sem1).wait()
            compute(rows1, h0 + 1)

            @pl.when(h0 + 3 < HW)
            def _():
                pltpu.async_copy(feat_hbm.at[adj_i.at[h0 + 3]], rows1, sem1)

            return carry

        lax.fori_loop(0, HW // 2, body, 0)
        pltpu.sync_copy(sc_v, p_out.at[pl.ds(wid * VW4, VW4)])

    return k(l, adjl, features)


def _tc_select_call(partials, adjl):
    blk = 256

    def body(p_ref, adj_ref, o_ref):
        jj = lax.broadcasted_iota(jnp.int32, (N * L, N), 0)
        nn2 = lax.broadcasted_iota(jnp.int32, (N * L, N), 1)
        M = (jj // L == nn2).astype(jnp.float32)
        x = jnp.dot(p_ref[:], M, preferred_element_type=jnp.float32)
        x = jnp.maximum(x, 0.0)
        x = jnp.where(x == 0.0, jnp.float32(9999.0), x)
        xn = x[:, :, None]
        xm = x[:, None, :]
        ii = lax.broadcasted_iota(jnp.int32, (blk, N, N), 1)
        jj = lax.broadcasted_iota(jnp.int32, (blk, N, N), 2)
        cmp = (xm < xn) | ((xm == xn) & (jj < ii))
        rank = jnp.sum(cmp.astype(jnp.int32), axis=2)
        ss = lax.broadcasted_iota(jnp.int32, (blk, N, S), 2)
        oh = rank[:, :, None] == ss
        adj = adj_ref[:]
        o_ref[:] = jnp.sum(jnp.where(oh, adj[:, :, None], 0), axis=1)

    V4 = V // 4
    return pl.pallas_call(
        body,
        grid=(V4 // blk,),
        in_specs=[
            pl.BlockSpec((blk, N * L), lambda i: (i, 0)),
            pl.BlockSpec((blk, N), lambda i: (i, 0)),
        ],
        out_specs=pl.BlockSpec((blk, S), lambda i: (i, 0)),
        out_shape=jax.ShapeDtypeStruct((V4, S), jnp.int32),
    )(partials, adjl)


def kernel(ids, num_samples, adj_info, features, W, b):
    V4 = V // 4
    adj2 = _tc_pair_call(adj_info)
    samp = []
    for q in range(4):
        ids_q = lax.slice(ids, (q * V4,), ((q + 1) * V4,))
        vf_q, stage_q = _sc_gather_call(ids_q, adj2, features)
        adjl_q = _tc_halve_call(stage_q, ids_q.reshape(V4, 1))
        l_q = _tc_matmul_call(vf_q, W, b)
        p_q = _sc_scores_call(l_q, adjl_q, features)
        samp.append(_tc_select_call(p_q, adjl_q))
    sampled = jnp.concatenate(samp, axis=0)
    return sampled + (jnp.asarray(num_samples, jnp.int32) - jnp.int32(S))
